# baseline (speedup 1.0000x reference)
import functools

import jax
import jax.numpy as jnp
from jax import lax
from jax.experimental import pallas as pl
from jax.experimental.pallas import tpu as pltpu
from jax.experimental.pallas import tpu_sc as plsc

_L = 16
_ROWS = 8
_D = 64


def _sc_body(x_hbm, out_hbm, rows_v, res_v):
    cid = lax.axis_index("c")
    sid = lax.axis_index("s")
    wid = sid * 2 + cid

    @pl.when(wid == 0)
    def _():
        pltpu.sync_copy(x_hbm.at[pl.ds(0, _ROWS)], rows_v)
        lane = lax.iota(jnp.int32, _L)
        acc = jnp.zeros((_L,), jnp.int32)
        for row, skip_col in ((1, 2), (2, -1)):
            for chunk in range(_D // _L):
                v = rows_v[row, pl.ds(chunk * _L, _L)]
                neq = jnp.where(v != jnp.float32(1.0),
                                jnp.int32(1), jnp.int32(0))
                if 0 <= skip_col - chunk * _L < _L:
                    neq = jnp.where(lane != jnp.int32(skip_col - chunk * _L),
                                    neq, jnp.int32(0))
                acc = acc | neq
        res_v[...] = acc | jnp.int32(1)
        pltpu.sync_copy(res_v, out_hbm)


_sc_diff = functools.partial(
    pl.kernel,
    mesh=plsc.VectorSubcoreMesh(core_axis_name="c", subcore_axis_name="s",
                                num_cores=1),
    out_type=jax.ShapeDtypeStruct((_L,), jnp.int32),
    scratch_types=[
        pltpu.VMEM((_ROWS, _D), jnp.float32),
        pltpu.VMEM((_L,), jnp.int32),
    ],
)(_sc_body)


def kernel(x):
    out = _sc_diff(x)
    return out[0].astype(jnp.bool_)


# speedup vs baseline: 1.0144x; 1.0144x over previous
import functools

import jax
import jax.numpy as jnp
from jax import lax
from jax.experimental import pallas as pl
from jax.experimental.pallas import tpu as pltpu


def _tc_body(x_ref, o_ref):
    rows = x_ref[pl.ds(1, 2), :]
    neq = rows != jnp.float32(1.0)
    ri = lax.broadcasted_iota(jnp.int32, (2, 64), 0)
    ci = lax.broadcasted_iota(jnp.int32, (2, 64), 1)
    keep = jnp.logical_or(ri != 0, ci != 2)
    diff = jnp.any(jnp.logical_and(neq, keep))
    o_ref[0, 0] = diff.astype(jnp.int32) | 1


_tc_diff = pl.pallas_call(
    _tc_body,
    out_shape=jax.ShapeDtypeStruct((1, 1), jnp.int32),
    grid=(1,),
    in_specs=[pl.BlockSpec((8, 64), lambda i: (0, 0))],
    out_specs=pl.BlockSpec(memory_space=pltpu.SMEM),
)


def kernel(x):
    out = _tc_diff(x)
    return out[0, 0].astype(jnp.bool_)


# speedup vs baseline: 17.5740x; 17.3237x over previous
import functools

import jax
import jax.numpy as jnp
from jax import lax
from jax.experimental import pallas as pl
from jax.experimental.pallas import tpu as pltpu
from jax.experimental.pallas import tpu_sc as plsc

_L = 16
_ROWS = 8
_D = 64


def _sc_body(x_hbm, out_hbm, rows_v, res_v):
    cid = lax.axis_index("c")
    sid = lax.axis_index("s")
    wid = sid * 2 + cid

    @pl.when(wid == 0)
    def _():
        pltpu.sync_copy(x_hbm, rows_v)
        lane = lax.iota(jnp.int32, _L)
        acc = jnp.zeros((_L,), jnp.int32)
        for row, skip_col in ((1, 2), (2, -1)):
            for chunk in range(_D // _L):
                v = rows_v[row, pl.ds(chunk * _L, _L)]
                neq = jnp.where(v != jnp.float32(1.0),
                                jnp.int32(1), jnp.int32(0))
                if 0 <= skip_col - chunk * _L < _L:
                    neq = jnp.where(lane != jnp.int32(skip_col - chunk * _L),
                                    neq, jnp.int32(0))
                acc = acc | neq
        res_v[...] = acc | jnp.int32(1)
        pltpu.sync_copy(res_v, out_hbm)


_sc_diff = functools.partial(
    pl.kernel,
    mesh=plsc.VectorSubcoreMesh(core_axis_name="c", subcore_axis_name="s"),
    out_type=jax.ShapeDtypeStruct((_L,), jnp.int32),
    scratch_types=[
        pltpu.VMEM((_ROWS, _D), jnp.float32),
        pltpu.VMEM((_L,), jnp.int32),
    ],
)(_sc_body)


def kernel(x):
    out = _sc_diff(lax.slice(x, (0, 0), (_ROWS, _D)))
    return out[0].astype(jnp.bool_)


# speedup vs baseline: 20.3017x; 1.1552x over previous
import functools

import jax
import jax.numpy as jnp
from jax import lax
from jax.experimental import pallas as pl
from jax.experimental.pallas import tpu as pltpu
from jax.experimental.pallas import tpu_sc as plsc

_ROWS = 8
_D = 64


def _scs_body(x_hbm, out_hbm, rows_s, res_s):
    cid = lax.axis_index("c")

    @pl.when(cid == 0)
    def _():
        pltpu.sync_copy(x_hbm, rows_s)

        def scan_row(row, skip_col, acc):
            def body(c, a):
                v = rows_s[row, c]
                hit = jnp.logical_and(v != jnp.float32(1.0), c != skip_col)
                return a | hit.astype(jnp.int32)
            return lax.fori_loop(0, _D, body, acc)

        acc = scan_row(1, 2, jnp.int32(0))
        acc = scan_row(2, -1, acc)
        result = acc | jnp.int32(1)

        def fill(i, _):
            res_s[i] = result
            return 0

        lax.fori_loop(0, 16, fill, 0)
        pltpu.sync_copy(res_s, out_hbm)


_scs_diff = functools.partial(
    pl.kernel,
    mesh=plsc.ScalarSubcoreMesh(axis_name="c", num_cores=1),
    out_type=jax.ShapeDtypeStruct((16,), jnp.int32),
    scratch_types=[
        pltpu.SMEM((_ROWS, _D), jnp.float32),
        pltpu.SMEM((16,), jnp.int32),
    ],
)(_scs_body)


def kernel(x):
    out = _scs_diff(lax.slice(x, (0, 0), (_ROWS, _D)))
    return out[0].astype(jnp.bool_)
